# Initial kernel scaffold; baseline (speedup 1.0000x reference)
import functools

import jax
import jax.numpy as jnp
from jax import lax
from jax.experimental import pallas as pl
from jax.experimental.pallas import tpu as pltpu
from jax.experimental.pallas import tpu_sc as plsc

_N = 10000
_E = 320000
_D_IN = 128
_ENC1 = 64
_ENC2 = 32
_OUT = 64
_R = 4

_BLK = 128
_TILES = 32
_ROWS = 2560
_EP = _ROWS * _BLK
_TROWS = _ROWS // _TILES
_AROWS = _R * _N
_TSLICE = _AROWS // 16
_ZR = 125

_HIGH = lax.Precision.HIGHEST


def _encoder_body(x_ref, w1_ref, b1_ref, w2_ref, b2_ref, o_ref):
    h1 = jnp.maximum(jnp.dot(x_ref[...], w1_ref[...], precision=_HIGH)
                     + b1_ref[...], 0.0)
    o_ref[...] = jnp.maximum(jnp.dot(h1, w2_ref[...], precision=_HIGH)
                             + b2_ref[...], 0.0)


def _encoder(x, w1, b1, w2, b2):
    blk = 1000
    grid = _N // blk
    return pl.pallas_call(
        _encoder_body,
        grid=(grid,),
        in_specs=[
            pl.BlockSpec((blk, _D_IN), lambda i: (i, 0)),
            pl.BlockSpec((_D_IN, _ENC1), lambda i: (0, 0)),
            pl.BlockSpec((1, _ENC1), lambda i: (0, 0)),
            pl.BlockSpec((_ENC1, _ENC2), lambda i: (0, 0)),
            pl.BlockSpec((1, _ENC2), lambda i: (0, 0)),
        ],
        out_specs=pl.BlockSpec((blk, _ENC2), lambda i: (i, 0)),
        out_shape=jax.ShapeDtypeStruct((_N, _ENC2), jnp.float32),
    )(x, w1, b1.reshape(1, _ENC1), w2, b2.reshape(1, _ENC2))


def _sc_edge_aggregate(h, src2, sidx2, norm2):
    mesh = plsc.VectorSubcoreMesh(core_axis_name="c", subcore_axis_name="s")

    @functools.partial(
        pl.kernel,
        out_type=jax.ShapeDtypeStruct((2, _AROWS, _ENC2), jnp.float32),
        mesh=mesh,
        scratch_types=[
            pltpu.VMEM((_TROWS, _BLK), jnp.int32),
            pltpu.VMEM((_TROWS, _BLK), jnp.int32),
            pltpu.VMEM((_TROWS, _BLK), jnp.float32),
            pltpu.VMEM((_BLK, _ENC2), jnp.float32),
            pltpu.VMEM((_BLK, _ENC2), jnp.float32),
            pltpu.VMEM((_ZR, _ENC2), jnp.float32),
            pltpu.VMEM_SHARED((_AROWS, _ENC2), jnp.float32),
            pltpu.SemaphoreType.DMA,
            pltpu.SemaphoreType.DMA,
        ],
    )
    def sc_kernel(h_hbm, src_hbm, sidx_hbm, norm_hbm, out_hbm,
                  srcv, sidxv, normv, rows0, rows1, zbuf, a_sh, sem0, sem1):
        cid = lax.axis_index("c")
        sid = lax.axis_index("s")
        tbase = (cid * 16 + sid) * _TROWS

        @pl.loop(0, _ZR)
        def _(i):
            zbuf[i, pl.ds(0, 16)] = jnp.zeros((16,), jnp.float32)
            zbuf[i, pl.ds(16, 16)] = jnp.zeros((16,), jnp.float32)

        abase = sid * _TSLICE

        @pl.loop(0, _TSLICE, step=_ZR)
        def _(q):
            pltpu.sync_copy(zbuf, a_sh.at[pl.ds(abase + q, _ZR)])

        pltpu.sync_copy(src_hbm.at[pl.ds(tbase, _TROWS)], srcv)
        pltpu.sync_copy(sidx_hbm.at[pl.ds(tbase, _TROWS)], sidxv)
        pltpu.sync_copy(norm_hbm.at[pl.ds(tbase, _TROWS)], normv)

        plsc.subcore_barrier()

        def scale(rows, k):
            @pl.loop(0, _BLK)
            def _(i):
                n = normv[k, i]
                rows[i, pl.ds(0, 16)] = rows[i, pl.ds(0, 16)] * n
                rows[i, pl.ds(16, 16)] = rows[i, pl.ds(16, 16)] * n

        pltpu.async_copy(h_hbm.at[srcv.at[0]], rows0, sem0)

        @pl.loop(0, _TROWS, step=2)
        def _(k):
            pltpu.make_async_copy(h_hbm.at[srcv.at[k]], rows0, sem0).wait()
            pltpu.async_copy(h_hbm.at[srcv.at[k + 1]], rows1, sem1)
            scale(rows0, k)
            pltpu.sync_copy(rows0, a_sh.at[sidxv.at[k]], add=True)
            pltpu.make_async_copy(h_hbm.at[srcv.at[k + 1]], rows1, sem1).wait()

            @pl.when(k + 2 < _TROWS)
            def _():
                pltpu.async_copy(h_hbm.at[srcv.at[k + 2]], rows0, sem0)

            scale(rows1, k + 1)
            pltpu.sync_copy(rows1, a_sh.at[sidxv.at[k + 1]], add=True)

        plsc.subcore_barrier()
        pltpu.sync_copy(a_sh.at[pl.ds(abase, _TSLICE)],
                        out_hbm.at[cid, pl.ds(abase, _TSLICE)])

    return sc_kernel(h, src2, sidx2, norm2)


def _final_body(a_ref, h_ref, ws_ref, b_ref, o_ref):
    acc = jnp.dot(h_ref[...], ws_ref[2 * _R], precision=_HIGH)
    for k in range(2 * _R):
        acc = acc + jnp.dot(a_ref[k], ws_ref[k], precision=_HIGH)
    o_ref[...] = jnp.maximum(acc + b_ref[...], 0.0)


def _finalize(a8, h, ws, bias):
    blk = 1000
    grid = _N // blk
    return pl.pallas_call(
        _final_body,
        grid=(grid,),
        in_specs=[
            pl.BlockSpec((2 * _R, blk, _ENC2), lambda i: (0, i, 0)),
            pl.BlockSpec((blk, _ENC2), lambda i: (i, 0)),
            pl.BlockSpec((2 * _R + 1, _ENC2, _OUT), lambda i: (0, 0, 0)),
            pl.BlockSpec((1, _OUT), lambda i: (0, 0)),
        ],
        out_specs=pl.BlockSpec((blk, _OUT), lambda i: (i, 0)),
        out_shape=jax.ShapeDtypeStruct((_N, _OUT), jnp.float32),
    )(a8, h, ws, bias.reshape(1, _OUT))


def kernel(node_features, edge_index, edgetypes, norm, W1, b1, W2, b2,
           Wrel, Wloop, bias):
    h = _encoder(node_features, W1, b1, W2, b2)

    src = edge_index[0]
    dst = edge_index[1]
    sidx = edgetypes * _N + dst
    pad = _EP - _E
    pidx = jnp.arange(pad, dtype=jnp.int32)
    src_p = jnp.concatenate([src, pidx % _N]).reshape(_ROWS, _BLK)
    sidx_p = jnp.concatenate([sidx, pidx % _AROWS]).reshape(_ROWS, _BLK)
    norm_p = jnp.concatenate(
        [norm[:, 0], jnp.zeros((pad,), jnp.float32)]).reshape(_ROWS, _BLK)

    a_parts = _sc_edge_aggregate(h, src_p, sidx_p, norm_p)
    a8 = a_parts.reshape(2 * _R, _N, _ENC2)

    ws = jnp.concatenate([Wrel, Wrel, Wloop[None]], axis=0)
    return _finalize(a8, h, ws, bias)


# speedup vs baseline: 12.1538x; 12.1538x over previous
import functools

import jax
import jax.numpy as jnp
from jax import lax
from jax.experimental import pallas as pl
from jax.experimental.pallas import tpu as pltpu
from jax.experimental.pallas import tpu_sc as plsc

_N = 10000
_E = 320000
_D_IN = 128
_ENC1 = 64
_ENC2 = 32
_OUT = 64
_R = 4

_BLK = 128
_TILES = 32
_ROWS = 2560
_EP = _ROWS * _BLK
_TROWS = _ROWS // _TILES
_AROWS = _R * _N
_APAD = 40960
_TSLICE = _APAD // 16
_ZR = 128


def _encoder_body(x_ref, w1_ref, b1_ref, w2_ref, b2_ref, o_ref):
    h1 = jnp.maximum(jnp.dot(x_ref[...], w1_ref[...]) + b1_ref[...], 0.0)
    o_ref[...] = jnp.maximum(jnp.dot(h1, w2_ref[...]) + b2_ref[...], 0.0)


def _encoder(x, w1, b1, w2, b2):
    blk = 1000
    grid = _N // blk
    return pl.pallas_call(
        _encoder_body,
        grid=(grid,),
        in_specs=[
            pl.BlockSpec((blk, _D_IN), lambda i: (i, 0)),
            pl.BlockSpec((_D_IN, _ENC1), lambda i: (0, 0)),
            pl.BlockSpec((1, _ENC1), lambda i: (0, 0)),
            pl.BlockSpec((_ENC1, _ENC2), lambda i: (0, 0)),
            pl.BlockSpec((1, _ENC2), lambda i: (0, 0)),
        ],
        out_specs=pl.BlockSpec((blk, _ENC2), lambda i: (i, 0)),
        out_shape=jax.ShapeDtypeStruct((_N, _ENC2), jnp.float32),
    )(x, w1, b1.reshape(1, _ENC1), w2, b2.reshape(1, _ENC2))


def _sc_edge_aggregate(h, src2, sidx2, norm2):
    mesh = plsc.VectorSubcoreMesh(core_axis_name="c", subcore_axis_name="s")

    @functools.partial(
        pl.kernel,
        out_type=jax.ShapeDtypeStruct((2, _APAD, _ENC2), jnp.float32),
        mesh=mesh,
        compiler_params=pltpu.CompilerParams(use_tc_tiling_on_sc=False),
        scratch_types=[
            pltpu.VMEM((_TROWS, _BLK), jnp.int32),
            pltpu.VMEM((_TROWS, _BLK), jnp.int32),
            pltpu.VMEM((_TROWS, _BLK), jnp.float32),
            pltpu.VMEM((_BLK, _ENC2), jnp.float32),
            pltpu.VMEM((_BLK, _ENC2), jnp.float32),
            pltpu.VMEM((_ZR, _ENC2), jnp.float32),
            pltpu.VMEM_SHARED((_APAD, _ENC2), jnp.float32),
            pltpu.SemaphoreType.DMA,
            pltpu.SemaphoreType.DMA,
        ],
    )
    def sc_kernel(h_hbm, src_hbm, sidx_hbm, norm_hbm, out_hbm,
                  srcv, sidxv, normv, rows0, rows1, zbuf, a_sh, sem0, sem1):
        cid = lax.axis_index("c")
        sid = lax.axis_index("s")
        tbase = (cid * 16 + sid) * _TROWS

        @pl.loop(0, _ZR)
        def _(i):
            zbuf[i, pl.ds(0, 16)] = jnp.zeros((16,), jnp.float32)
            zbuf[i, pl.ds(16, 16)] = jnp.zeros((16,), jnp.float32)

        abase = sid * _TSLICE

        @pl.loop(0, _TSLICE, step=_ZR)
        def _(q):
            pltpu.sync_copy(zbuf, a_sh.at[pl.ds(abase + q, _ZR)])

        pltpu.sync_copy(src_hbm.at[pl.ds(tbase, _TROWS)], srcv)
        pltpu.sync_copy(sidx_hbm.at[pl.ds(tbase, _TROWS)], sidxv)
        pltpu.sync_copy(norm_hbm.at[pl.ds(tbase, _TROWS)], normv)

        plsc.subcore_barrier()

        def scale(rows, k):
            @pl.loop(0, _BLK, step=16)
            def _(g):
                nv = normv[k, pl.ds(g, 16)]
                for j in range(16):
                    n = nv[j]
                    rows[g + j, pl.ds(0, 16)] = rows[g + j, pl.ds(0, 16)] * n
                    rows[g + j, pl.ds(16, 16)] = rows[g + j, pl.ds(16, 16)] * n

        pltpu.async_copy(h_hbm.at[srcv.at[0]], rows0, sem0)

        @pl.loop(0, _TROWS, step=2)
        def _(k):
            pltpu.make_async_copy(h_hbm.at[srcv.at[k]], rows0, sem0).wait()
            pltpu.async_copy(h_hbm.at[srcv.at[k + 1]], rows1, sem1)
            scale(rows0, k)
            pltpu.sync_copy(rows0, a_sh.at[sidxv.at[k]], add=True)
            pltpu.make_async_copy(h_hbm.at[srcv.at[k + 1]], rows1, sem1).wait()

            @pl.when(k + 2 < _TROWS)
            def _():
                pltpu.async_copy(h_hbm.at[srcv.at[k + 2]], rows0, sem0)

            scale(rows1, k + 1)
            pltpu.sync_copy(rows1, a_sh.at[sidxv.at[k + 1]], add=True)

        plsc.subcore_barrier()
        pltpu.sync_copy(a_sh.at[pl.ds(abase, _TSLICE)],
                        out_hbm.at[cid, pl.ds(abase, _TSLICE)])

    return sc_kernel(h, src2, sidx2, norm2)


def _final_body(a_ref, h_ref, wstk_ref, wloop_ref, b_ref, o_ref):
    s = a_ref[0] + a_ref[1]
    acc = jnp.dot(s, wstk_ref[...]) + jnp.dot(h_ref[...], wloop_ref[...])
    o_ref[...] = jnp.maximum(acc + b_ref[...], 0.0)


def _finalize(a2, h, wstk, wloop, bias):
    blk = 1000
    grid = _N // blk
    return pl.pallas_call(
        _final_body,
        grid=(grid,),
        in_specs=[
            pl.BlockSpec((2, blk, 4 * _ENC2), lambda i: (0, i, 0)),
            pl.BlockSpec((blk, _ENC2), lambda i: (i, 0)),
            pl.BlockSpec((4 * _ENC2, _OUT), lambda i: (0, 0)),
            pl.BlockSpec((_ENC2, _OUT), lambda i: (0, 0)),
            pl.BlockSpec((1, _OUT), lambda i: (0, 0)),
        ],
        out_specs=pl.BlockSpec((blk, _OUT), lambda i: (i, 0)),
        out_shape=jax.ShapeDtypeStruct((_N, _OUT), jnp.float32),
    )(a2, h, wstk, wloop, bias.reshape(1, _OUT))


def kernel(node_features, edge_index, edgetypes, norm, W1, b1, W2, b2,
           Wrel, Wloop, bias):
    h = _encoder(node_features, W1, b1, W2, b2)

    src = edge_index[0]
    dst = edge_index[1]
    sidx = 4 * dst + edgetypes
    pad = _EP - _E
    pidx = jnp.arange(pad, dtype=jnp.int32)
    src_p = jnp.concatenate([src, pidx % _N]).reshape(_ROWS, _BLK)
    sidx_p = jnp.concatenate([sidx, pidx % _APAD]).reshape(_ROWS, _BLK)
    norm_p = jnp.concatenate(
        [norm[:, 0], jnp.zeros((pad,), jnp.float32)]).reshape(_ROWS, _BLK)

    a_parts = _sc_edge_aggregate(h, src_p, sidx_p, norm_p)
    a2 = a_parts.reshape(2, _APAD // 4, 4 * _ENC2)

    return _finalize(a2, h, Wrel.reshape(_R * _ENC2, _OUT), Wloop, bias)


# speedup vs baseline: 13.0062x; 1.0701x over previous
import functools

import jax
import jax.numpy as jnp
from jax import lax
from jax.experimental import pallas as pl
from jax.experimental.pallas import tpu as pltpu
from jax.experimental.pallas import tpu_sc as plsc

_N = 10000
_E = 320000
_D_IN = 128
_ENC1 = 64
_ENC2 = 32
_OUT = 64
_R = 4

_BLK = 128
_TILES = 32
_ROWS = 2560
_EP = _ROWS * _BLK
_TROWS = _ROWS // _TILES
_AROWS = _R * _N
_APAD = 40960
_TSLICE = _APAD // 16
_ZR = 128


def _sidx_body(d_ref, e_ref, o_ref):
    o_ref[...] = 4 * d_ref[...] + e_ref[...]


def _sidx_prep(dst2, et2):
    blkr = 160
    return pl.pallas_call(
        _sidx_body,
        grid=(_ROWS // blkr,),
        in_specs=[
            pl.BlockSpec((blkr, _BLK), lambda i: (i, 0)),
            pl.BlockSpec((blkr, _BLK), lambda i: (i, 0)),
        ],
        out_specs=pl.BlockSpec((blkr, _BLK), lambda i: (i, 0)),
        out_shape=jax.ShapeDtypeStruct((_ROWS, _BLK), jnp.int32),
    )(dst2, et2)


def _encoder_body(x_ref, w1_ref, b1_ref, w2_ref, b2_ref, o_ref):
    h1 = jnp.maximum(jnp.dot(x_ref[...], w1_ref[...]) + b1_ref[...], 0.0)
    o_ref[...] = jnp.maximum(jnp.dot(h1, w2_ref[...]) + b2_ref[...], 0.0)


def _encoder(x, w1, b1, w2, b2):
    blk = 1000
    grid = _N // blk
    return pl.pallas_call(
        _encoder_body,
        grid=(grid,),
        in_specs=[
            pl.BlockSpec((blk, _D_IN), lambda i: (i, 0)),
            pl.BlockSpec((_D_IN, _ENC1), lambda i: (0, 0)),
            pl.BlockSpec((1, _ENC1), lambda i: (0, 0)),
            pl.BlockSpec((_ENC1, _ENC2), lambda i: (0, 0)),
            pl.BlockSpec((1, _ENC2), lambda i: (0, 0)),
        ],
        out_specs=pl.BlockSpec((blk, _ENC2), lambda i: (i, 0)),
        out_shape=jax.ShapeDtypeStruct((_N, _ENC2), jnp.float32),
    )(x, w1, b1.reshape(1, _ENC1), w2, b2.reshape(1, _ENC2))


def _sc_edge_aggregate(h, src2, sidx2, norm2):
    mesh = plsc.VectorSubcoreMesh(core_axis_name="c", subcore_axis_name="s")

    @functools.partial(
        pl.kernel,
        out_type=jax.ShapeDtypeStruct((2, _APAD, _ENC2), jnp.float32),
        mesh=mesh,
        compiler_params=pltpu.CompilerParams(use_tc_tiling_on_sc=False),
        scratch_types=[
            pltpu.VMEM((_TROWS, _BLK), jnp.int32),
            pltpu.VMEM((_TROWS, _BLK), jnp.int32),
            pltpu.VMEM((_TROWS, _BLK), jnp.float32),
            pltpu.VMEM((_BLK, _ENC2), jnp.float32),
            pltpu.VMEM((_BLK, _ENC2), jnp.float32),
            pltpu.VMEM((_ZR, _ENC2), jnp.float32),
            pltpu.VMEM_SHARED((_APAD, _ENC2), jnp.float32),
            pltpu.SemaphoreType.DMA,
            pltpu.SemaphoreType.DMA,
            pltpu.SemaphoreType.DMA,
            pltpu.SemaphoreType.DMA,
        ],
    )
    def sc_kernel(h_hbm, src_hbm, sidx_hbm, norm_hbm, out_hbm,
                  srcv, sidxv, normv, rows0, rows1, zbuf, a_sh,
                  sem0, sem1, ssem0, ssem1):
        cid = lax.axis_index("c")
        sid = lax.axis_index("s")
        tbase = (cid * 16 + sid) * _TROWS

        d1 = pltpu.async_copy(src_hbm.at[pl.ds(tbase, _TROWS)], srcv, ssem0)
        d2 = pltpu.async_copy(sidx_hbm.at[pl.ds(tbase, _TROWS)], sidxv, ssem0)
        d3 = pltpu.async_copy(norm_hbm.at[pl.ds(tbase, _TROWS)], normv, ssem1)

        @pl.loop(0, _ZR)
        def _(i):
            zbuf[i, pl.ds(0, 16)] = jnp.zeros((16,), jnp.float32)
            zbuf[i, pl.ds(16, 16)] = jnp.zeros((16,), jnp.float32)

        abase = sid * _TSLICE

        @pl.loop(0, _TSLICE, step=_ZR)
        def _(q):
            pltpu.sync_copy(zbuf, a_sh.at[pl.ds(abase + q, _ZR)])

        d1.wait()
        d2.wait()
        d3.wait()
        plsc.subcore_barrier()

        def scale(rows, k):
            @pl.loop(0, _BLK, step=16)
            def _(g):
                nv = normv[k, pl.ds(g, 16)]
                for j in range(16):
                    n = nv[j]
                    rows[g + j, pl.ds(0, 16)] = rows[g + j, pl.ds(0, 16)] * n
                    rows[g + j, pl.ds(16, 16)] = rows[g + j, pl.ds(16, 16)] * n

        pltpu.async_copy(h_hbm.at[srcv.at[0]], rows0, sem0)
        pltpu.async_copy(h_hbm.at[srcv.at[1]], rows1, sem1)

        @pl.loop(0, _TROWS, step=2)
        def _(k):
            pltpu.make_async_copy(h_hbm.at[srcv.at[k]], rows0, sem0).wait()
            scale(rows0, k)
            pltpu.async_copy(rows0, a_sh.at[sidxv.at[k]], ssem0, add=True)

            pltpu.make_async_copy(h_hbm.at[srcv.at[k + 1]], rows1, sem1).wait()
            scale(rows1, k + 1)
            pltpu.async_copy(rows1, a_sh.at[sidxv.at[k + 1]], ssem1, add=True)

            pltpu.make_async_copy(rows0, a_sh.at[sidxv.at[k]], ssem0).wait()

            @pl.when(k + 2 < _TROWS)
            def _():
                pltpu.async_copy(h_hbm.at[srcv.at[k + 2]], rows0, sem0)

            pltpu.make_async_copy(rows1, a_sh.at[sidxv.at[k + 1]], ssem1).wait()

            @pl.when(k + 3 < _TROWS)
            def _():
                pltpu.async_copy(h_hbm.at[srcv.at[k + 3]], rows1, sem1)

        plsc.subcore_barrier()
        pltpu.sync_copy(a_sh.at[pl.ds(abase, _TSLICE)],
                        out_hbm.at[cid, pl.ds(abase, _TSLICE)])

    return sc_kernel(h, src2, sidx2, norm2)


def _final_body(a_ref, h_ref, wstk_ref, wloop_ref, b_ref, o_ref):
    s = a_ref[0] + a_ref[1]
    acc = jnp.dot(s, wstk_ref[...]) + jnp.dot(h_ref[...], wloop_ref[...])
    o_ref[...] = jnp.maximum(acc + b_ref[...], 0.0)


def _finalize(a2, h, wstk, wloop, bias):
    blk = 2000
    grid = _N // blk
    return pl.pallas_call(
        _final_body,
        grid=(grid,),
        in_specs=[
            pl.BlockSpec((2, blk, 4 * _ENC2), lambda i: (0, i, 0)),
            pl.BlockSpec((blk, _ENC2), lambda i: (i, 0)),
            pl.BlockSpec((4 * _ENC2, _OUT), lambda i: (0, 0)),
            pl.BlockSpec((_ENC2, _OUT), lambda i: (0, 0)),
            pl.BlockSpec((1, _OUT), lambda i: (0, 0)),
        ],
        out_specs=pl.BlockSpec((blk, _OUT), lambda i: (i, 0)),
        out_shape=jax.ShapeDtypeStruct((_N, _OUT), jnp.float32),
    )(a2, h, wstk, wloop, bias.reshape(1, _OUT))


def kernel(node_features, edge_index, edgetypes, norm, W1, b1, W2, b2,
           Wrel, Wloop, bias):
    h = _encoder(node_features, W1, b1, W2, b2)

    pad = _EP - _E
    src_p = jnp.pad(edge_index[0], (0, pad), mode="wrap").reshape(_ROWS, _BLK)
    dst_p = jnp.pad(edge_index[1], (0, pad), mode="wrap").reshape(_ROWS, _BLK)
    et_p = jnp.pad(edgetypes, (0, pad), mode="wrap").reshape(_ROWS, _BLK)
    norm_p = jnp.pad(norm.reshape(-1), (0, pad)).reshape(_ROWS, _BLK)
    sidx_p = _sidx_prep(dst_p, et_p)

    a_parts = _sc_edge_aggregate(h, src_p, sidx_p, norm_p)
    a2 = a_parts.reshape(2, _APAD // 4, 4 * _ENC2)

    return _finalize(a2, h, Wrel.reshape(_R * _ENC2, _OUT), Wloop, bias)


# speedup vs baseline: 15.1277x; 1.1631x over previous
import functools

import jax
import jax.numpy as jnp
from jax import lax
from jax.experimental import pallas as pl
from jax.experimental.pallas import tpu as pltpu
from jax.experimental.pallas import tpu_sc as plsc

_N = 10000
_E = 320000
_D_IN = 128
_ENC1 = 64
_ENC2 = 32
_OUT = 64
_R = 4

_BLK = 128
_TILES = 32
_ROWS = 2560
_EP = _ROWS * _BLK
_TROWS = _ROWS // _TILES
_AROWS = _R * _N
_APAD = 40960
_TSLICE = _APAD // 16
_ZR = 128


_EB = _ROWS // 10
_ER = _E // _BLK


def _encoder_body(x_ref, w1_ref, b1_ref, w2_ref, b2_ref, eit_ref, et_ref,
                  nm_ref, o_ref, src_ref, sidx_ref, nrm_ref):
    h1 = jnp.maximum(jnp.dot(x_ref[...], w1_ref[...]) + b1_ref[...], 0.0)
    o_ref[...] = jnp.maximum(jnp.dot(h1, w2_ref[...]) + b2_ref[...], 0.0)

    i = pl.program_id(0)
    row = lax.broadcasted_iota(jnp.int32, (_EB, _BLK), 0) + i * _EB
    col = lax.broadcasted_iota(jnp.int32, (_EB, _BLK), 1)
    valid = row < _ER
    padv = (row & 63) * _BLK + col
    src_ref[...] = jnp.where(valid, eit_ref[:, 0, :], padv)
    sidx_ref[...] = jnp.where(valid, 4 * eit_ref[:, 1, :] + et_ref[...], padv)
    nrm_ref[...] = jnp.where(valid, nm_ref[...], 0.0)


def _encoder(x, w1, b1, w2, b2, eit, et2, nm2):
    blk = 1000
    grid = _N // blk
    return pl.pallas_call(
        _encoder_body,
        grid=(grid,),
        in_specs=[
            pl.BlockSpec((blk, _D_IN), lambda i: (i, 0)),
            pl.BlockSpec((_D_IN, _ENC1), lambda i: (0, 0)),
            pl.BlockSpec((1, _ENC1), lambda i: (0, 0)),
            pl.BlockSpec((_ENC1, _ENC2), lambda i: (0, 0)),
            pl.BlockSpec((1, _ENC2), lambda i: (0, 0)),
            pl.BlockSpec((_EB, 2, _BLK), lambda i: (i, 0, 0)),
            pl.BlockSpec((_EB, _BLK), lambda i: (i, 0)),
            pl.BlockSpec((_EB, _BLK), lambda i: (i, 0)),
        ],
        out_specs=[
            pl.BlockSpec((blk, _ENC2), lambda i: (i, 0)),
            pl.BlockSpec((_EB, _BLK), lambda i: (i, 0)),
            pl.BlockSpec((_EB, _BLK), lambda i: (i, 0)),
            pl.BlockSpec((_EB, _BLK), lambda i: (i, 0)),
        ],
        out_shape=[
            jax.ShapeDtypeStruct((_N, _ENC2), jnp.float32),
            jax.ShapeDtypeStruct((_ROWS, _BLK), jnp.int32),
            jax.ShapeDtypeStruct((_ROWS, _BLK), jnp.int32),
            jax.ShapeDtypeStruct((_ROWS, _BLK), jnp.float32),
        ],
    )(x, w1, b1.reshape(1, _ENC1), w2, b2.reshape(1, _ENC2), eit, et2, nm2)


def _sc_edge_aggregate(h, src2, sidx2, norm2):
    mesh = plsc.VectorSubcoreMesh(core_axis_name="c", subcore_axis_name="s")

    @functools.partial(
        pl.kernel,
        out_type=jax.ShapeDtypeStruct((2, _APAD, _ENC2), jnp.float32),
        mesh=mesh,
        compiler_params=pltpu.CompilerParams(use_tc_tiling_on_sc=False),
        scratch_types=[
            pltpu.VMEM((_TROWS, _BLK), jnp.int32),
            pltpu.VMEM((_TROWS, _BLK), jnp.int32),
            pltpu.VMEM((_TROWS, _BLK), jnp.float32),
            pltpu.VMEM((_BLK, _ENC2), jnp.float32),
            pltpu.VMEM((_BLK, _ENC2), jnp.float32),
            pltpu.VMEM((_ZR, _ENC2), jnp.float32),
            pltpu.VMEM_SHARED((_APAD, _ENC2), jnp.float32),
            pltpu.SemaphoreType.DMA,
            pltpu.SemaphoreType.DMA,
            pltpu.SemaphoreType.DMA,
            pltpu.SemaphoreType.DMA,
        ],
    )
    def sc_kernel(h_hbm, src_hbm, sidx_hbm, norm_hbm, out_hbm,
                  srcv, sidxv, normv, rows0, rows1, zbuf, a_sh,
                  sem0, sem1, ssem0, ssem1):
        cid = lax.axis_index("c")
        sid = lax.axis_index("s")
        tbase = (cid * 16 + sid) * _TROWS

        d1 = pltpu.async_copy(src_hbm.at[pl.ds(tbase, _TROWS)], srcv, ssem0)
        d2 = pltpu.async_copy(sidx_hbm.at[pl.ds(tbase, _TROWS)], sidxv, ssem0)
        d3 = pltpu.async_copy(norm_hbm.at[pl.ds(tbase, _TROWS)], normv, ssem1)

        @pl.loop(0, _ZR)
        def _(i):
            zbuf[i, pl.ds(0, 16)] = jnp.zeros((16,), jnp.float32)
            zbuf[i, pl.ds(16, 16)] = jnp.zeros((16,), jnp.float32)

        abase = sid * _TSLICE

        @pl.loop(0, _TSLICE, step=_ZR)
        def _(q):
            pltpu.sync_copy(zbuf, a_sh.at[pl.ds(abase + q, _ZR)])

        d1.wait()
        d2.wait()
        d3.wait()
        plsc.subcore_barrier()

        def scale(rows, k):
            @pl.loop(0, _BLK, step=16)
            def _(g):
                nv = normv[k, pl.ds(g, 16)]
                for j in range(16):
                    n = nv[j]
                    rows[g + j, pl.ds(0, 16)] = rows[g + j, pl.ds(0, 16)] * n
                    rows[g + j, pl.ds(16, 16)] = rows[g + j, pl.ds(16, 16)] * n

        pltpu.async_copy(h_hbm.at[srcv.at[0]], rows0, sem0)
        pltpu.async_copy(h_hbm.at[srcv.at[1]], rows1, sem1)

        @pl.loop(0, _TROWS, step=2)
        def _(k):
            pltpu.make_async_copy(h_hbm.at[srcv.at[k]], rows0, sem0).wait()
            scale(rows0, k)
            pltpu.async_copy(rows0, a_sh.at[sidxv.at[k]], ssem0, add=True)

            pltpu.make_async_copy(h_hbm.at[srcv.at[k + 1]], rows1, sem1).wait()
            scale(rows1, k + 1)
            pltpu.async_copy(rows1, a_sh.at[sidxv.at[k + 1]], ssem1, add=True)

            pltpu.make_async_copy(rows0, a_sh.at[sidxv.at[k]], ssem0).wait()

            @pl.when(k + 2 < _TROWS)
            def _():
                pltpu.async_copy(h_hbm.at[srcv.at[k + 2]], rows0, sem0)

            pltpu.make_async_copy(rows1, a_sh.at[sidxv.at[k + 1]], ssem1).wait()

            @pl.when(k + 3 < _TROWS)
            def _():
                pltpu.async_copy(h_hbm.at[srcv.at[k + 3]], rows1, sem1)

        plsc.subcore_barrier()
        pltpu.sync_copy(a_sh.at[pl.ds(abase, _TSLICE)],
                        out_hbm.at[cid, pl.ds(abase, _TSLICE)])

    return sc_kernel(h, src2, sidx2, norm2)


def _final_body(a_ref, h_ref, wstk_ref, wloop_ref, b_ref, o_ref):
    s = a_ref[0] + a_ref[1]
    acc = jnp.dot(s, wstk_ref[...]) + jnp.dot(h_ref[...], wloop_ref[...])
    o_ref[...] = jnp.maximum(acc + b_ref[...], 0.0)


def _finalize(a2, h, wstk, wloop, bias):
    blk = 2000
    grid = _N // blk
    return pl.pallas_call(
        _final_body,
        grid=(grid,),
        in_specs=[
            pl.BlockSpec((2, blk, 4 * _ENC2), lambda i: (0, i, 0)),
            pl.BlockSpec((blk, _ENC2), lambda i: (i, 0)),
            pl.BlockSpec((4 * _ENC2, _OUT), lambda i: (0, 0)),
            pl.BlockSpec((_ENC2, _OUT), lambda i: (0, 0)),
            pl.BlockSpec((1, _OUT), lambda i: (0, 0)),
        ],
        out_specs=pl.BlockSpec((blk, _OUT), lambda i: (i, 0)),
        out_shape=jax.ShapeDtypeStruct((_N, _OUT), jnp.float32),
    )(a2, h, wstk, wloop, bias.reshape(1, _OUT))


def kernel(node_features, edge_index, edgetypes, norm, W1, b1, W2, b2,
           Wrel, Wloop, bias):
    eit = edge_index.reshape(2, _ER, _BLK).transpose(1, 0, 2)
    et2 = edgetypes.reshape(_ER, _BLK)
    nm2 = norm.reshape(_ER, _BLK)
    h, src_p, sidx_p, norm_p = _encoder(node_features, W1, b1, W2, b2,
                                        eit, et2, nm2)

    a_parts = _sc_edge_aggregate(h, src_p, sidx_p, norm_p)
    a2 = a_parts.reshape(2, _APAD // 4, 4 * _ENC2)

    return _finalize(a2, h, Wrel.reshape(_R * _ENC2, _OUT), Wloop, bias)
